# baseline (speedup 1.0000x reference)
import functools

import jax
import jax.numpy as jnp
from jax import lax
from jax.experimental import pallas as pl
from jax.experimental.pallas import tpu as pltpu

_TAU_H = 2.0
_ALPHA_X = 0.1
_ALPHA_H = 0.05


def _round_up(n, m):
    return ((n + m - 1) // m) * m


def _quant8(w):
    amax = jnp.max(jnp.abs(w))
    s = jnp.exp2(jnp.floor(jnp.log2(192.0 / jnp.maximum(amax, 1e-30))))
    return (w * s).astype(jnp.float8_e4m3fn), (1.0 / s).astype(jnp.float32)


def _rnn_kernel(scal_ref, x_ref, h0_ref, u0_ref, wr_ref, g_ref, m_ref,
                wos_ref, wot_ref, bo_ref, gv_ref, err_ref,
                h_scr, u_scr, q_scr, p_scr, *, time_chunk, unroll):
    chunk = pl.program_id(0)

    @pl.when(chunk == 0)
    def _():
        h_scr[...] = h0_ref[...]
        u_scr[...] = u0_ref[...]

    C = time_chunk
    B, S = h_scr.shape
    O = x_ref.shape[-1]
    inv_tau = 1.0 / _TAU_H
    f8 = jnp.float8_e4m3fn
    bf16 = jnp.bfloat16
    f32 = jnp.float32

    c_rec = scal_ref[0] * inv_tau
    c_g = scal_ref[1]

    xmat = jnp.reshape(x_ref[...], (C * B, O)).astype(bf16)
    xw = jnp.dot(xmat, wos_ref[...], preferred_element_type=f32)
    q_scr[...] = jnp.reshape(
        jnp.broadcast_to(gv_ref[...], (C * B, S)) - xw, (C, B, S)).astype(bf16)

    wr = wr_ref[...]
    g = g_ref[...]
    m = m_ref[...]

    def step(t, carry):
        h, u = carry
        a = jnp.tanh(h)
        rec = jnp.dot(a.astype(f8), wr, preferred_element_type=f32)
        h_prior = (1.0 - inv_tau) * h + inv_tau * u + c_rec * rec
        p = jnp.tanh(h_prior)
        p_scr[t] = p.astype(bf16)
        e = c_g * jnp.dot(p.astype(f8), g, preferred_element_type=f32) \
            + q_scr[t].astype(f32)
        d = (1.0 - p * p) * e
        h_new = h_prior - d
        u_new = u - jnp.dot(d.astype(bf16), m, preferred_element_type=f32)
        return h_new, u_new

    h_fin, u_fin = lax.fori_loop(0, time_chunk, step,
                                 (h_scr[...], u_scr[...]), unroll=unroll)
    h_scr[...] = h_fin
    u_scr[...] = u_fin

    pmat = jnp.reshape(p_scr[...], (C * B, S))
    xpred = jnp.dot(pmat, wot_ref[...], preferred_element_type=f32)
    bo = jnp.broadcast_to(bo_ref[...], (C * B, O))
    err_ref[...] = jnp.reshape(
        xpred + bo - jnp.reshape(x_ref[...], (C * B, O)), (C, B, O))


def kernel(x, c_init, h_init, w_o, b_o, w_r, b_r, w_c):
    seq_len, batch, output_dim = x.shape
    states_dim = w_r.shape[0]
    f32 = jnp.float32
    bf16 = jnp.bfloat16

    B_p = _round_up(max(batch, 1), 8)
    O_p = _round_up(output_dim, 128)
    S_p = _round_up(states_dim, 128)

    time_chunk = min(seq_len, 32)
    T_p = _round_up(seq_len, time_chunk)
    n_chunks = T_p // time_chunk
    unroll = 32

    x_p = jnp.zeros((T_p, B_p, O_p), f32)
    x_p = x_p.at[:seq_len, :batch, :output_dim].set(x.astype(f32))
    h0 = jnp.zeros((B_p, S_p), f32).at[:batch, :states_dim].set(h_init.astype(f32))

    w_o32 = w_o.astype(f32)
    w_c32 = w_c.astype(f32)

    u_init = c_init.astype(f32) @ w_c32.T + b_r.astype(f32)[None, :]
    u0 = jnp.zeros((B_p, S_p), f32).at[:batch, :states_dim].set(u_init)

    wr_full = jnp.zeros((S_p, S_p), f32).at[:states_dim, :states_dim].set(
        w_r.astype(f32).T)
    g_full = jnp.zeros((S_p, S_p), f32).at[:states_dim, :states_dim].set(
        _ALPHA_X * (w_o32.T @ w_o32))
    wr8, inv_sr = _quant8(wr_full)
    g8, inv_sg = _quant8(g_full)
    scal = jnp.stack([inv_sr, inv_sg])

    m_mat = jnp.zeros((S_p, S_p), f32).at[:states_dim, :states_dim].set(
        _ALPHA_H * (w_c32 @ w_c32.T)).astype(bf16)
    wos = jnp.zeros((O_p, S_p), f32).at[:output_dim, :states_dim].set(
        _ALPHA_X * w_o32).astype(bf16)
    wot = jnp.zeros((S_p, O_p), f32).at[:states_dim, :output_dim].set(
        w_o32.T).astype(bf16)
    bo = jnp.zeros((1, O_p), f32).at[0, :output_dim].set(b_o.astype(f32))
    gvec = jnp.zeros((1, S_p), f32).at[0, :states_dim].set(
        _ALPHA_X * (b_o.astype(f32) @ w_o32))

    body = functools.partial(_rnn_kernel, time_chunk=time_chunk, unroll=unroll)

    def _const_spec(shape):
        return pl.BlockSpec(shape, lambda i, s, _n=len(shape): (0,) * _n)

    errors_p = pl.pallas_call(
        body,
        out_shape=jax.ShapeDtypeStruct((T_p, B_p, O_p), f32),
        grid_spec=pltpu.PrefetchScalarGridSpec(
            num_scalar_prefetch=1,
            grid=(n_chunks,),
            in_specs=[
                pl.BlockSpec((time_chunk, B_p, O_p), lambda i, s: (i, 0, 0)),
                _const_spec((B_p, S_p)),
                _const_spec((B_p, S_p)),
                _const_spec((S_p, S_p)),
                _const_spec((S_p, S_p)),
                _const_spec((S_p, S_p)),
                _const_spec((O_p, S_p)),
                _const_spec((S_p, O_p)),
                _const_spec((1, O_p)),
                _const_spec((1, S_p)),
            ],
            out_specs=pl.BlockSpec((time_chunk, B_p, O_p),
                                   lambda i, s: (i, 0, 0)),
            scratch_shapes=[
                pltpu.VMEM((B_p, S_p), f32),
                pltpu.VMEM((B_p, S_p), f32),
                pltpu.VMEM((time_chunk, B_p, S_p), bf16),
                pltpu.VMEM((time_chunk, B_p, S_p), bf16),
            ],
        ),
        compiler_params=pltpu.CompilerParams(
            dimension_semantics=("arbitrary",)),
    )(scal, x_p, h0, u0, wr8, g8, m_mat, wos, wot, bo, gvec)

    return errors_p[:seq_len, :batch, :output_dim]


# speedup vs baseline: 1.0253x; 1.0253x over previous
import functools

import jax
import jax.numpy as jnp
from jax import lax
from jax.experimental import pallas as pl
from jax.experimental.pallas import tpu as pltpu

_TAU_H = 2.0
_ALPHA_X = 0.1
_ALPHA_H = 0.05


def _round_up(n, m):
    return ((n + m - 1) // m) * m


def _quant8(w):
    amax = jnp.max(jnp.abs(w))
    s = jnp.exp2(jnp.floor(jnp.log2(192.0 / jnp.maximum(amax, 1e-30))))
    return (w * s).astype(jnp.float8_e4m3fn), (1.0 / s).astype(jnp.float32)


def _rnn_kernel(scal_ref, x_ref, h0_ref, u0_ref, wr_ref, g_ref, m_ref,
                wos_ref, wot_ref, bo_ref, gv_ref, err_ref,
                h_scr, u_scr, q_scr, p_scr, *, time_chunk, unroll):
    chunk = pl.program_id(0)

    @pl.when(chunk == 0)
    def _():
        h_scr[...] = h0_ref[...]
        u_scr[...] = u0_ref[...]

    C = time_chunk
    B, S = h_scr.shape
    O = x_ref.shape[-1]
    inv_tau = 1.0 / _TAU_H
    f8 = jnp.float8_e4m3fn
    bf16 = jnp.bfloat16
    f32 = jnp.float32

    c_rec = scal_ref[0] * inv_tau
    c_g = scal_ref[1]

    xmat = jnp.reshape(x_ref[...], (C * B, O)).astype(bf16)
    xw = jnp.dot(xmat, wos_ref[...], preferred_element_type=f32)
    q_scr[...] = jnp.reshape(
        jnp.broadcast_to(gv_ref[...], (C * B, S)) - xw, (C, B, S)).astype(bf16)

    wr = wr_ref[...]
    g = g_ref[...]
    m = m_ref[...]

    def step(t, carry):
        h, u = carry
        a = jnp.tanh(h)
        rec = jnp.dot(a.astype(f8), wr, preferred_element_type=f32)
        h_prior = (1.0 - inv_tau) * h + inv_tau * u + c_rec * rec
        p = jnp.tanh(h_prior)
        p_scr[t] = p.astype(bf16)
        e = c_g * jnp.dot(p.astype(f8), g, preferred_element_type=f32) \
            + q_scr[t].astype(f32)
        d = (1.0 - p * p) * e
        h_new = h_prior - d
        u_new = u - jnp.dot(d.astype(bf16), m, preferred_element_type=f32)
        return h_new, u_new

    h_fin, u_fin = lax.fori_loop(0, time_chunk, step,
                                 (h_scr[...], u_scr[...]), unroll=unroll)
    h_scr[...] = h_fin
    u_scr[...] = u_fin

    pmat = jnp.reshape(p_scr[...], (C * B, S))
    xpred = jnp.dot(pmat, wot_ref[...], preferred_element_type=f32)
    bo = jnp.broadcast_to(bo_ref[...], (C * B, O))
    err_ref[...] = jnp.reshape(
        xpred + bo - jnp.reshape(x_ref[...], (C * B, O)), (C, B, O))


def kernel(x, c_init, h_init, w_o, b_o, w_r, b_r, w_c):
    seq_len, batch, output_dim = x.shape
    states_dim = w_r.shape[0]
    f32 = jnp.float32
    bf16 = jnp.bfloat16

    B_p = _round_up(max(batch, 1), 8)
    O_p = _round_up(output_dim, 128)
    S_p = _round_up(states_dim, 128)

    time_chunk = min(seq_len, 128)
    T_p = _round_up(seq_len, time_chunk)
    n_chunks = T_p // time_chunk
    unroll = 8

    x_p = jnp.zeros((T_p, B_p, O_p), f32)
    x_p = x_p.at[:seq_len, :batch, :output_dim].set(x.astype(f32))
    h0 = jnp.zeros((B_p, S_p), f32).at[:batch, :states_dim].set(h_init.astype(f32))

    w_o32 = w_o.astype(f32)
    w_c32 = w_c.astype(f32)

    u_init = c_init.astype(f32) @ w_c32.T + b_r.astype(f32)[None, :]
    u0 = jnp.zeros((B_p, S_p), f32).at[:batch, :states_dim].set(u_init)

    wr_full = jnp.zeros((S_p, S_p), f32).at[:states_dim, :states_dim].set(
        w_r.astype(f32).T)
    g_full = jnp.zeros((S_p, S_p), f32).at[:states_dim, :states_dim].set(
        _ALPHA_X * (w_o32.T @ w_o32))
    wr8, inv_sr = _quant8(wr_full)
    g8, inv_sg = _quant8(g_full)
    scal = jnp.stack([inv_sr, inv_sg])

    m_mat = jnp.zeros((S_p, S_p), f32).at[:states_dim, :states_dim].set(
        _ALPHA_H * (w_c32 @ w_c32.T)).astype(bf16)
    wos = jnp.zeros((O_p, S_p), f32).at[:output_dim, :states_dim].set(
        _ALPHA_X * w_o32).astype(bf16)
    wot = jnp.zeros((S_p, O_p), f32).at[:states_dim, :output_dim].set(
        w_o32.T).astype(bf16)
    bo = jnp.zeros((1, O_p), f32).at[0, :output_dim].set(b_o.astype(f32))
    gvec = jnp.zeros((1, S_p), f32).at[0, :states_dim].set(
        _ALPHA_X * (b_o.astype(f32) @ w_o32))

    body = functools.partial(_rnn_kernel, time_chunk=time_chunk, unroll=unroll)

    def _const_spec(shape):
        return pl.BlockSpec(shape, lambda i, s, _n=len(shape): (0,) * _n)

    errors_p = pl.pallas_call(
        body,
        out_shape=jax.ShapeDtypeStruct((T_p, B_p, O_p), f32),
        grid_spec=pltpu.PrefetchScalarGridSpec(
            num_scalar_prefetch=1,
            grid=(n_chunks,),
            in_specs=[
                pl.BlockSpec((time_chunk, B_p, O_p), lambda i, s: (i, 0, 0)),
                _const_spec((B_p, S_p)),
                _const_spec((B_p, S_p)),
                _const_spec((S_p, S_p)),
                _const_spec((S_p, S_p)),
                _const_spec((S_p, S_p)),
                _const_spec((O_p, S_p)),
                _const_spec((S_p, O_p)),
                _const_spec((1, O_p)),
                _const_spec((1, S_p)),
            ],
            out_specs=pl.BlockSpec((time_chunk, B_p, O_p),
                                   lambda i, s: (i, 0, 0)),
            scratch_shapes=[
                pltpu.VMEM((B_p, S_p), f32),
                pltpu.VMEM((B_p, S_p), f32),
                pltpu.VMEM((time_chunk, B_p, S_p), bf16),
                pltpu.VMEM((time_chunk, B_p, S_p), bf16),
            ],
        ),
        compiler_params=pltpu.CompilerParams(
            dimension_semantics=("arbitrary",)),
    )(scal, x_p, h0, u0, wr8, g8, m_mat, wos, wot, bo, gvec)

    return errors_p[:seq_len, :batch, :output_dim]


# speedup vs baseline: 1.0654x; 1.0390x over previous
import functools

import jax
import jax.numpy as jnp
from jax import lax
from jax.experimental import pallas as pl
from jax.experimental.pallas import tpu as pltpu

_TAU_H = 2.0
_ALPHA_X = 0.1
_ALPHA_H = 0.05


def _round_up(n, m):
    return ((n + m - 1) // m) * m


def _pow2_scale(target, amax):
    return jnp.exp2(jnp.floor(jnp.log2(target / jnp.maximum(amax, 1e-30))))


def _quant8(w):
    s = _pow2_scale(192.0, jnp.max(jnp.abs(w)))
    return (w * s).astype(jnp.float8_e4m3fn), (1.0 / s).astype(jnp.float32)


def _rnn_kernel(scal_ref, x_ref, h0_ref, u0_ref, wr_ref, g_ref, m_ref,
                wos_ref, wot_ref, bo_ref, gv_ref, err_ref,
                h_scr, u_scr, q_scr, p_scr, *, time_chunk, unroll):
    chunk = pl.program_id(0)

    @pl.when(chunk == 0)
    def _():
        h_scr[...] = h0_ref[...]
        u_scr[...] = u0_ref[...]

    C = time_chunk
    B, S = h_scr.shape
    O = x_ref.shape[-1]
    inv_tau = 1.0 / _TAU_H
    f8 = jnp.float8_e4m3fn
    bf16 = jnp.bfloat16
    f32 = jnp.float32

    c_rec = scal_ref[0] * inv_tau
    c_g = scal_ref[1]
    s_d = scal_ref[2]
    c_m = scal_ref[3]

    xmat = jnp.reshape(x_ref[...], (C * B, O)).astype(bf16)
    xw = jnp.dot(xmat, wos_ref[...], preferred_element_type=f32)
    q_scr[...] = jnp.reshape(
        jnp.broadcast_to(gv_ref[...], (C * B, S)) - xw, (C, B, S)).astype(bf16)

    wr = wr_ref[...]
    g = g_ref[...]
    m = m_ref[...]

    def step(t, carry):
        h, u = carry
        a = jnp.tanh(h)
        rec = jnp.dot(a.astype(f8), wr, preferred_element_type=f32)
        h_prior = (1.0 - inv_tau) * h + inv_tau * u + c_rec * rec
        p = jnp.tanh(h_prior)
        p_scr[t] = p.astype(bf16)
        e = c_g * jnp.dot(p.astype(f8), g, preferred_element_type=f32) \
            + q_scr[t].astype(f32)
        d = (1.0 - p * p) * e
        h_new = h_prior - d
        d8 = (s_d * d).astype(f8)
        u_new = u - c_m * jnp.dot(d8, m, preferred_element_type=f32)
        return h_new, u_new

    h_fin, u_fin = lax.fori_loop(0, time_chunk, step,
                                 (h_scr[...], u_scr[...]), unroll=unroll)
    h_scr[...] = h_fin
    u_scr[...] = u_fin

    pmat = jnp.reshape(p_scr[...], (C * B, S))
    xpred = jnp.dot(pmat, wot_ref[...], preferred_element_type=f32)
    bo = jnp.broadcast_to(bo_ref[...], (C * B, O))
    err_ref[...] = jnp.reshape(
        xpred + bo - jnp.reshape(x_ref[...], (C * B, O)), (C, B, O))


def kernel(x, c_init, h_init, w_o, b_o, w_r, b_r, w_c):
    seq_len, batch, output_dim = x.shape
    states_dim = w_r.shape[0]
    f32 = jnp.float32
    bf16 = jnp.bfloat16

    B_p = _round_up(max(batch, 1), 8)
    O_p = _round_up(output_dim, 128)
    S_p = _round_up(states_dim, 128)

    time_chunk = min(seq_len, 64)
    T_p = _round_up(seq_len, time_chunk)
    n_chunks = T_p // time_chunk
    unroll = 8

    x_p = jnp.zeros((T_p, B_p, O_p), f32)
    x_p = x_p.at[:seq_len, :batch, :output_dim].set(x.astype(f32))
    h0 = jnp.zeros((B_p, S_p), f32).at[:batch, :states_dim].set(h_init.astype(f32))

    w_o32 = w_o.astype(f32)
    w_c32 = w_c.astype(f32)

    u_init = c_init.astype(f32) @ w_c32.T + b_r.astype(f32)[None, :]
    u0 = jnp.zeros((B_p, S_p), f32).at[:batch, :states_dim].set(u_init)

    wr_full = jnp.zeros((S_p, S_p), f32).at[:states_dim, :states_dim].set(
        w_r.astype(f32).T)
    g_full = jnp.zeros((S_p, S_p), f32).at[:states_dim, :states_dim].set(
        _ALPHA_X * (w_o32.T @ w_o32))
    m_full = jnp.zeros((S_p, S_p), f32).at[:states_dim, :states_dim].set(
        _ALPHA_H * (w_c32 @ w_c32.T))
    wr8, inv_sr = _quant8(wr_full)
    g8, inv_sg = _quant8(g_full)
    m8, inv_sm = _quant8(m_full)

    xpred_max = jnp.max(jnp.sum(jnp.abs(w_o32), axis=1)) + jnp.max(
        jnp.abs(b_o.astype(f32)))
    err_max = xpred_max + jnp.max(jnp.abs(x.astype(f32)))
    d_max = _ALPHA_X * err_max * jnp.max(jnp.sum(jnp.abs(w_o32), axis=0))
    s_d = _pow2_scale(96.0, d_max)
    inv_sd = 1.0 / s_d

    scal = jnp.stack([inv_sr, inv_sg, s_d, inv_sd * inv_sm])

    wos = jnp.zeros((O_p, S_p), f32).at[:output_dim, :states_dim].set(
        _ALPHA_X * w_o32).astype(bf16)
    wot = jnp.zeros((S_p, O_p), f32).at[:states_dim, :output_dim].set(
        w_o32.T).astype(bf16)
    bo = jnp.zeros((1, O_p), f32).at[0, :output_dim].set(b_o.astype(f32))
    gvec = jnp.zeros((1, S_p), f32).at[0, :states_dim].set(
        _ALPHA_X * (b_o.astype(f32) @ w_o32))

    body = functools.partial(_rnn_kernel, time_chunk=time_chunk, unroll=unroll)

    def _const_spec(shape):
        return pl.BlockSpec(shape, lambda i, s, _n=len(shape): (0,) * _n)

    errors_p = pl.pallas_call(
        body,
        out_shape=jax.ShapeDtypeStruct((T_p, B_p, O_p), f32),
        grid_spec=pltpu.PrefetchScalarGridSpec(
            num_scalar_prefetch=1,
            grid=(n_chunks,),
            in_specs=[
                pl.BlockSpec((time_chunk, B_p, O_p), lambda i, s: (i, 0, 0)),
                _const_spec((B_p, S_p)),
                _const_spec((B_p, S_p)),
                _const_spec((S_p, S_p)),
                _const_spec((S_p, S_p)),
                _const_spec((S_p, S_p)),
                _const_spec((O_p, S_p)),
                _const_spec((S_p, O_p)),
                _const_spec((1, O_p)),
                _const_spec((1, S_p)),
            ],
            out_specs=pl.BlockSpec((time_chunk, B_p, O_p),
                                   lambda i, s: (i, 0, 0)),
            scratch_shapes=[
                pltpu.VMEM((B_p, S_p), f32),
                pltpu.VMEM((B_p, S_p), f32),
                pltpu.VMEM((time_chunk, B_p, S_p), bf16),
                pltpu.VMEM((time_chunk, B_p, S_p), bf16),
            ],
        ),
        compiler_params=pltpu.CompilerParams(
            dimension_semantics=("arbitrary",)),
    )(scal, x_p, h0, u0, wr8, g8, m8, wos, wot, bo, gvec)

    return errors_p[:seq_len, :batch, :output_dim]


# speedup vs baseline: 1.0831x; 1.0166x over previous
import functools

import jax
import jax.numpy as jnp
from jax import lax
from jax.experimental import pallas as pl
from jax.experimental.pallas import tpu as pltpu

_TAU_H = 2.0
_ALPHA_X = 0.1
_ALPHA_H = 0.05


def _round_up(n, m):
    return ((n + m - 1) // m) * m


def _pow2_scale(target, amax):
    return jnp.exp2(jnp.floor(jnp.log2(target / jnp.maximum(amax, 1e-30))))


def _quant8(w):
    s = _pow2_scale(192.0, jnp.max(jnp.abs(w)))
    return (w * s).astype(jnp.float8_e4m3fn), (1.0 / s).astype(jnp.float32)


def _rnn_kernel(scal_ref, x_ref, h0_ref, u0_ref, wr_ref, g_ref, m_ref,
                wos_ref, wot_ref, bo_ref, gv_ref, err_ref,
                h_scr, u_scr, q_scr, p_scr, *, time_chunk, unroll):
    chunk = pl.program_id(0)

    @pl.when(chunk == 0)
    def _():
        h_scr[...] = h0_ref[...]
        u_scr[...] = u0_ref[...]

    C = time_chunk
    B, S = h_scr.shape
    O = x_ref.shape[-1]
    inv_tau = 1.0 / _TAU_H
    f8 = jnp.float8_e4m3fn
    bf16 = jnp.bfloat16
    f32 = jnp.float32

    c_rec = scal_ref[0] * inv_tau
    c_g = scal_ref[1]
    s_d = scal_ref[2]
    c_m = scal_ref[3]

    xmat = jnp.reshape(x_ref[...], (C * B, O)).astype(bf16)
    xw = jnp.dot(xmat, wos_ref[...], preferred_element_type=f32)
    q_scr[...] = jnp.reshape(
        jnp.broadcast_to(gv_ref[...], (C * B, S)) - xw, (C, B, S)).astype(bf16)

    wr = wr_ref[...]
    g = g_ref[...]
    m = m_ref[...]

    def step(t, carry):
        h, u = carry
        a = jnp.tanh(h)
        rec = jnp.dot(a.astype(f8), wr, preferred_element_type=f32)
        h_prior = (1.0 - inv_tau) * h + inv_tau * u + c_rec * rec
        p = jnp.tanh(h_prior)
        p_scr[t] = p.astype(bf16)
        e = c_g * jnp.dot(p.astype(f8), g, preferred_element_type=f32) \
            + q_scr[t].astype(f32)
        d = (1.0 - p * p) * e
        h_new = h_prior - d
        d8 = (s_d * d).astype(f8)
        u_new = u - c_m * jnp.dot(d8, m, preferred_element_type=f32)
        return h_new, u_new

    h_fin, u_fin = lax.fori_loop(0, time_chunk, step,
                                 (h_scr[...], u_scr[...]), unroll=unroll)
    h_scr[...] = h_fin
    u_scr[...] = u_fin

    pmat = jnp.reshape(p_scr[...], (C * B, S))
    xpred = jnp.dot(pmat, wot_ref[...], preferred_element_type=f32)
    bo = jnp.broadcast_to(bo_ref[...], (C * B, O))
    err_ref[...] = jnp.reshape(
        xpred + bo - jnp.reshape(x_ref[...], (C * B, O)), (C, B, O))


def kernel(x, c_init, h_init, w_o, b_o, w_r, b_r, w_c):
    seq_len, batch, output_dim = x.shape
    states_dim = w_r.shape[0]
    f32 = jnp.float32
    bf16 = jnp.bfloat16

    B_p = _round_up(max(batch, 1), 8)
    O_p = _round_up(output_dim, 128)
    S_p = _round_up(states_dim, 128)

    time_chunk = min(seq_len, 64)
    T_p = _round_up(seq_len, time_chunk)
    n_chunks = T_p // time_chunk
    unroll = 16

    x_p = jnp.zeros((T_p, B_p, O_p), f32)
    x_p = x_p.at[:seq_len, :batch, :output_dim].set(x.astype(f32))
    h0 = jnp.zeros((B_p, S_p), f32).at[:batch, :states_dim].set(h_init.astype(f32))

    w_o32 = w_o.astype(f32)
    w_c32 = w_c.astype(f32)

    u_init = c_init.astype(f32) @ w_c32.T + b_r.astype(f32)[None, :]
    u0 = jnp.zeros((B_p, S_p), f32).at[:batch, :states_dim].set(u_init)

    wr_full = jnp.zeros((S_p, S_p), f32).at[:states_dim, :states_dim].set(
        w_r.astype(f32).T)
    g_full = jnp.zeros((S_p, S_p), f32).at[:states_dim, :states_dim].set(
        _ALPHA_X * (w_o32.T @ w_o32))
    m_full = jnp.zeros((S_p, S_p), f32).at[:states_dim, :states_dim].set(
        _ALPHA_H * (w_c32 @ w_c32.T))
    wr8, inv_sr = _quant8(wr_full)
    g8, inv_sg = _quant8(g_full)
    m8, inv_sm = _quant8(m_full)

    xpred_max = jnp.max(jnp.sum(jnp.abs(w_o32), axis=1)) + jnp.max(
        jnp.abs(b_o.astype(f32)))
    err_max = xpred_max + jnp.max(jnp.abs(x.astype(f32)))
    d_max = _ALPHA_X * err_max * jnp.max(jnp.sum(jnp.abs(w_o32), axis=0))
    s_d = _pow2_scale(96.0, d_max)
    inv_sd = 1.0 / s_d

    scal = jnp.stack([inv_sr, inv_sg, s_d, inv_sd * inv_sm])

    wos = jnp.zeros((O_p, S_p), f32).at[:output_dim, :states_dim].set(
        _ALPHA_X * w_o32).astype(bf16)
    wot = jnp.zeros((S_p, O_p), f32).at[:states_dim, :output_dim].set(
        w_o32.T).astype(bf16)
    bo = jnp.zeros((1, O_p), f32).at[0, :output_dim].set(b_o.astype(f32))
    gvec = jnp.zeros((1, S_p), f32).at[0, :states_dim].set(
        _ALPHA_X * (b_o.astype(f32) @ w_o32))

    body = functools.partial(_rnn_kernel, time_chunk=time_chunk, unroll=unroll)

    def _const_spec(shape):
        return pl.BlockSpec(shape, lambda i, s, _n=len(shape): (0,) * _n)

    errors_p = pl.pallas_call(
        body,
        out_shape=jax.ShapeDtypeStruct((T_p, B_p, O_p), f32),
        grid_spec=pltpu.PrefetchScalarGridSpec(
            num_scalar_prefetch=1,
            grid=(n_chunks,),
            in_specs=[
                pl.BlockSpec((time_chunk, B_p, O_p), lambda i, s: (i, 0, 0)),
                _const_spec((B_p, S_p)),
                _const_spec((B_p, S_p)),
                _const_spec((S_p, S_p)),
                _const_spec((S_p, S_p)),
                _const_spec((S_p, S_p)),
                _const_spec((O_p, S_p)),
                _const_spec((S_p, O_p)),
                _const_spec((1, O_p)),
                _const_spec((1, S_p)),
            ],
            out_specs=pl.BlockSpec((time_chunk, B_p, O_p),
                                   lambda i, s: (i, 0, 0)),
            scratch_shapes=[
                pltpu.VMEM((B_p, S_p), f32),
                pltpu.VMEM((B_p, S_p), f32),
                pltpu.VMEM((time_chunk, B_p, S_p), bf16),
                pltpu.VMEM((time_chunk, B_p, S_p), bf16),
            ],
        ),
        compiler_params=pltpu.CompilerParams(
            dimension_semantics=("arbitrary",)),
    )(scal, x_p, h0, u0, wr8, g8, m8, wos, wot, bo, gvec)

    return errors_p[:seq_len, :batch, :output_dim]


# speedup vs baseline: 1.1158x; 1.0302x over previous
import functools

import jax
import jax.numpy as jnp
from jax import lax
from jax.experimental import pallas as pl
from jax.experimental.pallas import tpu as pltpu

_TAU_H = 2.0
_ALPHA_X = 0.1
_ALPHA_H = 0.05


def _round_up(n, m):
    return ((n + m - 1) // m) * m


def _pow2_scale(target, amax):
    return jnp.exp2(jnp.floor(jnp.log2(target / jnp.maximum(amax, 1e-30))))


def _quant8(w):
    s = _pow2_scale(192.0, jnp.max(jnp.abs(w)))
    return (w * s).astype(jnp.float8_e4m3fn), (1.0 / s).astype(jnp.float32)


def _rnn_kernel(scal_ref, x_ref, h0_ref, u0_ref, wr_ref, g_ref, m_ref,
                wos_ref, wot_ref, bo_ref, gv_ref, err_ref,
                h_scr, u_scr, d_scr, *, time_chunk, unroll):
    chunk = pl.program_id(0)

    @pl.when(chunk == 0)
    def _():
        h_scr[...] = h0_ref[...]
        u_scr[...] = u0_ref[...]
        d_scr[...] = jnp.zeros_like(d_scr)

    B, S = h_scr.shape
    inv_tau = 1.0 / _TAU_H
    f8 = jnp.float8_e4m3fn
    bf16 = jnp.bfloat16
    f32 = jnp.float32

    c_rec = scal_ref[0] * inv_tau
    c_g = scal_ref[1]
    s_d = scal_ref[2]
    c_m = scal_ref[3]
    c_o = scal_ref[4]

    wr = wr_ref[...]
    g = g_ref[...]
    m = m_ref[...]
    wos = wos_ref[...]
    wot = wot_ref[...]
    gv = gv_ref[...]
    bo = bo_ref[...]

    def step(t, carry):
        h, u, d8p = carry
        xt = x_ref[t]
        a = jnp.tanh(h)
        rec = jnp.dot(a.astype(f8), wr, preferred_element_type=f32)
        u_new = u - c_m * jnp.dot(d8p, m, preferred_element_type=f32)
        xw = jnp.dot(xt.astype(bf16), wos, preferred_element_type=f32)
        h_prior = (1.0 - inv_tau) * h + inv_tau * u_new + c_rec * rec
        p = jnp.tanh(h_prior)
        p8 = p.astype(f8)
        e = c_g * jnp.dot(p8, g, preferred_element_type=f32) + (gv - xw)
        err_ref[t] = c_o * jnp.dot(p8, wot, preferred_element_type=f32) \
            + (bo - xt)
        d = (1.0 - p * p) * e
        h_new = h_prior - d
        d8 = (s_d * d).astype(f8)
        return h_new, u_new, d8

    h_fin, u_fin, d8_fin = lax.fori_loop(
        0, time_chunk, step,
        (h_scr[...], u_scr[...], d_scr[...]), unroll=unroll)
    h_scr[...] = h_fin
    u_scr[...] = u_fin
    d_scr[...] = d8_fin


def kernel(x, c_init, h_init, w_o, b_o, w_r, b_r, w_c):
    seq_len, batch, output_dim = x.shape
    states_dim = w_r.shape[0]
    f32 = jnp.float32
    bf16 = jnp.bfloat16

    B_p = _round_up(max(batch, 1), 8)
    O_p = _round_up(output_dim, 128)
    S_p = _round_up(states_dim, 128)

    time_chunk = min(seq_len, 128)
    T_p = _round_up(seq_len, time_chunk)
    n_chunks = T_p // time_chunk
    unroll = 16

    x_p = jnp.zeros((T_p, B_p, O_p), f32)
    x_p = x_p.at[:seq_len, :batch, :output_dim].set(x.astype(f32))
    h0 = jnp.zeros((B_p, S_p), f32).at[:batch, :states_dim].set(h_init.astype(f32))

    w_o32 = w_o.astype(f32)
    w_c32 = w_c.astype(f32)

    u_init = c_init.astype(f32) @ w_c32.T + b_r.astype(f32)[None, :]
    u0 = jnp.zeros((B_p, S_p), f32).at[:batch, :states_dim].set(u_init)

    wr_full = jnp.zeros((S_p, S_p), f32).at[:states_dim, :states_dim].set(
        w_r.astype(f32).T)
    g_full = jnp.zeros((S_p, S_p), f32).at[:states_dim, :states_dim].set(
        _ALPHA_X * (w_o32.T @ w_o32))
    m_full = jnp.zeros((S_p, S_p), f32).at[:states_dim, :states_dim].set(
        _ALPHA_H * (w_c32 @ w_c32.T))
    wot_full = jnp.zeros((S_p, O_p), f32).at[:states_dim, :output_dim].set(
        w_o32.T)
    wr8, inv_sr = _quant8(wr_full)
    g8, inv_sg = _quant8(g_full)
    m8, inv_sm = _quant8(m_full)
    wot8, inv_so = _quant8(wot_full)

    xpred_max = jnp.max(jnp.sum(jnp.abs(w_o32), axis=1)) + jnp.max(
        jnp.abs(b_o.astype(f32)))
    err_max = xpred_max + jnp.max(jnp.abs(x.astype(f32)))
    d_max = _ALPHA_X * err_max * jnp.max(jnp.sum(jnp.abs(w_o32), axis=0))
    s_d = _pow2_scale(96.0, d_max)
    inv_sd = 1.0 / s_d

    scal = jnp.stack([inv_sr, inv_sg, s_d, inv_sd * inv_sm, inv_so])

    wos = jnp.zeros((O_p, S_p), f32).at[:output_dim, :states_dim].set(
        _ALPHA_X * w_o32).astype(bf16)
    bo = jnp.zeros((1, O_p), f32).at[0, :output_dim].set(b_o.astype(f32))
    gvec = jnp.zeros((1, S_p), f32).at[0, :states_dim].set(
        _ALPHA_X * (b_o.astype(f32) @ w_o32))

    body = functools.partial(_rnn_kernel, time_chunk=time_chunk, unroll=unroll)

    def _const_spec(shape):
        return pl.BlockSpec(shape, lambda i, s, _n=len(shape): (0,) * _n)

    errors_p = pl.pallas_call(
        body,
        out_shape=jax.ShapeDtypeStruct((T_p, B_p, O_p), f32),
        grid_spec=pltpu.PrefetchScalarGridSpec(
            num_scalar_prefetch=1,
            grid=(n_chunks,),
            in_specs=[
                pl.BlockSpec((time_chunk, B_p, O_p), lambda i, s: (i, 0, 0)),
                _const_spec((B_p, S_p)),
                _const_spec((B_p, S_p)),
                _const_spec((S_p, S_p)),
                _const_spec((S_p, S_p)),
                _const_spec((S_p, S_p)),
                _const_spec((O_p, S_p)),
                _const_spec((S_p, O_p)),
                _const_spec((1, O_p)),
                _const_spec((1, S_p)),
            ],
            out_specs=pl.BlockSpec((time_chunk, B_p, O_p),
                                   lambda i, s: (i, 0, 0)),
            scratch_shapes=[
                pltpu.VMEM((B_p, S_p), f32),
                pltpu.VMEM((B_p, S_p), f32),
                pltpu.VMEM((B_p, S_p), jnp.float8_e4m3fn),
            ],
        ),
        compiler_params=pltpu.CompilerParams(
            dimension_semantics=("arbitrary",)),
    )(scal, x_p, h0, u0, wr8, g8, m8, wos, wot8, bo, gvec)

    return errors_p[:seq_len, :batch, :output_dim]


# speedup vs baseline: 1.1224x; 1.0059x over previous
import functools

import jax
import jax.numpy as jnp
from jax import lax
from jax.experimental import pallas as pl
from jax.experimental.pallas import tpu as pltpu

_TAU_H = 2.0
_ALPHA_X = 0.1
_ALPHA_H = 0.05


def _round_up(n, m):
    return ((n + m - 1) // m) * m


def _pow2_scale(target, amax):
    return jnp.exp2(jnp.floor(jnp.log2(target / jnp.maximum(amax, 1e-30))))


def _quant8(w):
    s = _pow2_scale(192.0, jnp.max(jnp.abs(w)))
    return (w * s).astype(jnp.float8_e4m3fn), (1.0 / s).astype(jnp.float32)


def _rnn_kernel(scal_ref, x_ref, h0_ref, u0_ref, wr_ref, g_ref, m_ref,
                wos_ref, wot_ref, bo_ref, gv_ref, err_ref,
                h_scr, u_scr, d_scr, *, time_chunk, unroll):
    chunk = pl.program_id(0)

    @pl.when(chunk == 0)
    def _():
        h_scr[...] = h0_ref[...]
        u_scr[...] = u0_ref[...]
        d_scr[...] = jnp.zeros_like(d_scr)

    B, S = h_scr.shape
    inv_tau = 1.0 / _TAU_H
    f8 = jnp.float8_e4m3fn
    bf16 = jnp.bfloat16
    f32 = jnp.float32

    c_rec = scal_ref[0] * inv_tau
    c_g = scal_ref[1]
    s_d = scal_ref[2]
    c_m = scal_ref[3]
    c_o = scal_ref[4]

    wr = wr_ref[...]
    g = g_ref[...]
    m = m_ref[...]
    wos = wos_ref[...]
    wot = wot_ref[...]
    gv = gv_ref[...]
    bo = bo_ref[...]

    def step(t, carry):
        h, u, d8p = carry
        xt = x_ref[t]
        a = jnp.tanh(h)
        rec = jnp.dot(a.astype(f8), wr, preferred_element_type=f32)
        u_new = u - c_m * jnp.dot(d8p, m, preferred_element_type=f32)
        xw = jnp.dot(xt.astype(bf16), wos, preferred_element_type=f32)
        h_prior = (1.0 - inv_tau) * h + inv_tau * u_new + c_rec * rec
        p = jnp.tanh(h_prior)
        p8 = p.astype(f8)
        e = c_g * jnp.dot(p8, g, preferred_element_type=f32) + (gv - xw)
        err_ref[t] = c_o * jnp.dot(p8, wot, preferred_element_type=f32) \
            + (bo - xt)
        d = (1.0 - p * p) * e
        h_new = h_prior - d
        d8 = (s_d * d).astype(f8)
        return h_new, u_new, d8

    h_fin, u_fin, d8_fin = lax.fori_loop(
        0, time_chunk, step,
        (h_scr[...], u_scr[...], d_scr[...]), unroll=unroll)
    h_scr[...] = h_fin
    u_scr[...] = u_fin
    d_scr[...] = d8_fin


def kernel(x, c_init, h_init, w_o, b_o, w_r, b_r, w_c):
    seq_len, batch, output_dim = x.shape
    states_dim = w_r.shape[0]
    f32 = jnp.float32
    bf16 = jnp.bfloat16

    B_p = _round_up(max(batch, 1), 8)
    O_p = _round_up(output_dim, 128)
    S_p = _round_up(states_dim, 128)

    time_chunk = min(seq_len, 128)
    T_p = _round_up(seq_len, time_chunk)
    n_chunks = T_p // time_chunk
    unroll = 32

    x_p = jnp.zeros((T_p, B_p, O_p), f32)
    x_p = x_p.at[:seq_len, :batch, :output_dim].set(x.astype(f32))
    h0 = jnp.zeros((B_p, S_p), f32).at[:batch, :states_dim].set(h_init.astype(f32))

    w_o32 = w_o.astype(f32)
    w_c32 = w_c.astype(f32)

    u_init = c_init.astype(f32) @ w_c32.T + b_r.astype(f32)[None, :]
    u0 = jnp.zeros((B_p, S_p), f32).at[:batch, :states_dim].set(u_init)

    wr_full = jnp.zeros((S_p, S_p), f32).at[:states_dim, :states_dim].set(
        w_r.astype(f32).T)
    g_full = jnp.zeros((S_p, S_p), f32).at[:states_dim, :states_dim].set(
        _ALPHA_X * (w_o32.T @ w_o32))
    m_full = jnp.zeros((S_p, S_p), f32).at[:states_dim, :states_dim].set(
        _ALPHA_H * (w_c32 @ w_c32.T))
    wot_full = jnp.zeros((S_p, O_p), f32).at[:states_dim, :output_dim].set(
        w_o32.T)
    wr8, inv_sr = _quant8(wr_full)
    g8, inv_sg = _quant8(g_full)
    m8, inv_sm = _quant8(m_full)
    wot8, inv_so = _quant8(wot_full)

    xpred_max = jnp.max(jnp.sum(jnp.abs(w_o32), axis=1)) + jnp.max(
        jnp.abs(b_o.astype(f32)))
    err_max = xpred_max + jnp.max(jnp.abs(x.astype(f32)))
    d_max = _ALPHA_X * err_max * jnp.max(jnp.sum(jnp.abs(w_o32), axis=0))
    s_d = _pow2_scale(96.0, d_max)
    inv_sd = 1.0 / s_d

    scal = jnp.stack([inv_sr, inv_sg, s_d, inv_sd * inv_sm, inv_so])

    wos = jnp.zeros((O_p, S_p), f32).at[:output_dim, :states_dim].set(
        _ALPHA_X * w_o32).astype(bf16)
    bo = jnp.zeros((1, O_p), f32).at[0, :output_dim].set(b_o.astype(f32))
    gvec = jnp.zeros((1, S_p), f32).at[0, :states_dim].set(
        _ALPHA_X * (b_o.astype(f32) @ w_o32))

    body = functools.partial(_rnn_kernel, time_chunk=time_chunk, unroll=unroll)

    def _const_spec(shape):
        return pl.BlockSpec(shape, lambda i, s, _n=len(shape): (0,) * _n)

    errors_p = pl.pallas_call(
        body,
        out_shape=jax.ShapeDtypeStruct((T_p, B_p, O_p), f32),
        grid_spec=pltpu.PrefetchScalarGridSpec(
            num_scalar_prefetch=1,
            grid=(n_chunks,),
            in_specs=[
                pl.BlockSpec((time_chunk, B_p, O_p), lambda i, s: (i, 0, 0)),
                _const_spec((B_p, S_p)),
                _const_spec((B_p, S_p)),
                _const_spec((S_p, S_p)),
                _const_spec((S_p, S_p)),
                _const_spec((S_p, S_p)),
                _const_spec((O_p, S_p)),
                _const_spec((S_p, O_p)),
                _const_spec((1, O_p)),
                _const_spec((1, S_p)),
            ],
            out_specs=pl.BlockSpec((time_chunk, B_p, O_p),
                                   lambda i, s: (i, 0, 0)),
            scratch_shapes=[
                pltpu.VMEM((B_p, S_p), f32),
                pltpu.VMEM((B_p, S_p), f32),
                pltpu.VMEM((B_p, S_p), jnp.float8_e4m3fn),
            ],
        ),
        compiler_params=pltpu.CompilerParams(
            dimension_semantics=("arbitrary",)),
    )(scal, x_p, h0, u0, wr8, g8, m8, wos, wot8, bo, gvec)

    return errors_p[:seq_len, :batch, :output_dim]
